# Initial kernel scaffold; baseline (speedup 1.0000x reference)
import jax
import jax.numpy as jnp
from jax.experimental import pallas as pl


def kernel(x, edge_index, edge_weight, W1, b1, gamma, beta, W2, b2):
    raise NotImplementedError("write your pallas kernel here")


# speedup vs baseline: 2.5967x; 2.5967x over previous
import functools

import jax
import jax.numpy as jnp
from jax import lax
from jax.experimental import pallas as pl
from jax.experimental.pallas import tpu as pltpu
from jax.experimental.pallas import tpu_sc as plsc

N = 10000
NPAD = 10240
E = 320000
D = 128
EPS = 1e-5

NC = 2
NS = 16
NW = NC * NS
CH = 128
EPWP = 10240
EP = EPWP * NW
NCH = EPWP // CH
SPS = 10
SCH = NCH // SPS
RPT = NPAD // NS

_MESH = plsc.VectorSubcoreMesh(
    core_axis_name="c", subcore_axis_name="s", num_cores=NC, num_subcores=NS
)


def _spmv_partials(table, src5, dst4, w5):
    @functools.partial(
        pl.kernel,
        out_type=jax.ShapeDtypeStruct((NC * NPAD, D), jnp.float32),
        mesh=_MESH,
        scratch_types=[
            pltpu.VMEM((SCH, 1, CH), jnp.int32),
            pltpu.VMEM((NCH, 1, CH), jnp.int32),
            pltpu.VMEM((SCH, 1, CH), jnp.float32),
            pltpu.VMEM((CH, D), jnp.float32),
            pltpu.VMEM_SHARED((NPAD, D), jnp.float32),
            pltpu.SemaphoreType.DMA,
        ],
    )
    def k(table_hbm, src_hbm, dst_hbm, w_hbm, out_hbm,
          src_v, dst_v, w_v, rows_v, acc_sh, sem):
        cid = lax.axis_index("c")
        sid = lax.axis_index("s")
        wid = sid * NC + cid

        zero16 = jnp.zeros((16,), jnp.float32)

        def zrow(i, carry):
            for q in range(D // 16):
                rows_v[i, pl.ds(q * 16, 16)] = zero16
            return carry

        lax.fori_loop(0, CH, zrow, 0)
        for t in range(RPT // CH):
            pltpu.sync_copy(rows_v, acc_sh.at[pl.ds(sid * RPT + t * CH, CH)])

        pltpu.sync_copy(dst_hbm.at[wid], dst_v)

        plsc.subcore_barrier()

        for s in range(SPS):
            pltpu.sync_copy(src_hbm.at[wid, s], src_v)
            pltpu.sync_copy(w_hbm.at[wid, s], w_v)

            def chunk(jj, carry):
                pltpu.async_copy(
                    table_hbm.at[src_v.at[jj, 0]], rows_v, sem
                ).wait()

                def wmul(g, c2):
                    wv = w_v[jj, 0, pl.ds(g * 16, 16)]
                    for kk in range(16):
                        wr = wv[kk]
                        for q in range(D // 16):
                            rows_v[g * 16 + kk, pl.ds(q * 16, 16)] = (
                                rows_v[g * 16 + kk, pl.ds(q * 16, 16)] * wr
                            )
                    return c2

                lax.fori_loop(0, CH // 16, wmul, 0)
                pltpu.sync_copy(
                    rows_v, acc_sh.at[dst_v.at[s * SCH + jj, 0]], add=True
                )
                return carry

            lax.fori_loop(0, SCH, chunk, 0)

        plsc.subcore_barrier()
        pltpu.sync_copy(
            acc_sh.at[pl.ds(sid * RPT, RPT)],
            out_hbm.at[pl.ds(cid * NPAD + sid * RPT, RPT)],
        )

    return k(table, src5, dst4, w5)


_BLK = 2048
_GRID = NPAD // _BLK


def _combine(p):
    def body(a_ref, b_ref, o_ref):
        o_ref[...] = a_ref[...] + b_ref[...]

    return pl.pallas_call(
        body,
        grid=(_GRID,),
        in_specs=[
            pl.BlockSpec((_BLK, D), lambda i: (i, 0)),
            pl.BlockSpec((_BLK, D), lambda i: (i + _GRID, 0)),
        ],
        out_specs=pl.BlockSpec((_BLK, D), lambda i: (i, 0)),
        out_shape=jax.ShapeDtypeStruct((NPAD, D), jnp.float32),
    )(p, p)


def _cheb1(t0, t1, p2, W, b):
    def body(t0_ref, t1_ref, p2a_ref, p2b_ref, w_ref, b_ref, o_ref, st_ref):
        i = pl.program_id(0)
        t0v = t0_ref[...]
        t1v = t1_ref[...]
        t2v = 2.0 * (p2a_ref[...] + p2b_ref[...]) - t0v
        acc = (
            jnp.dot(t0v, w_ref[0], preferred_element_type=jnp.float32)
            + jnp.dot(t1v, w_ref[1], preferred_element_type=jnp.float32)
            + jnp.dot(t2v, w_ref[2], preferred_element_type=jnp.float32)
            + b_ref[...]
        )
        h = jnp.maximum(acc, 0.0)
        o_ref[...] = h
        rid = lax.broadcasted_iota(jnp.int32, (_BLK, 1), 0) + i * _BLK
        hm = jnp.where(rid < N, h, 0.0)
        ps = jnp.sum(hm, axis=0, keepdims=True)
        pq = jnp.sum(hm * hm, axis=0, keepdims=True)
        blk = jnp.concatenate([ps, pq], axis=0)

        @pl.when(i == 0)
        def _():
            st_ref[...] = blk

        @pl.when(i != 0)
        def _():
            st_ref[...] = st_ref[...] + blk

    return pl.pallas_call(
        body,
        grid=(_GRID,),
        in_specs=[
            pl.BlockSpec((_BLK, D), lambda i: (i, 0)),
            pl.BlockSpec((_BLK, D), lambda i: (i, 0)),
            pl.BlockSpec((_BLK, D), lambda i: (i, 0)),
            pl.BlockSpec((_BLK, D), lambda i: (i + _GRID, 0)),
            pl.BlockSpec((3, D, D), lambda i: (0, 0, 0)),
            pl.BlockSpec((1, D), lambda i: (0, 0)),
        ],
        out_specs=[
            pl.BlockSpec((_BLK, D), lambda i: (i, 0)),
            pl.BlockSpec((2, D), lambda i: (0, 0)),
        ],
        out_shape=[
            jax.ShapeDtypeStruct((NPAD, D), jnp.float32),
            jax.ShapeDtypeStruct((2, D), jnp.float32),
        ],
    )(t0, t1, p2, p2, W, b)


def _bn(h, stats, gamma, beta):
    def body(h_ref, st_ref, g_ref, be_ref, o_ref):
        mean = st_ref[0:1, :] * (1.0 / N)
        var = st_ref[1:2, :] * (1.0 / N) - mean * mean
        inv = lax.rsqrt(var + EPS)
        o_ref[...] = (h_ref[...] - mean) * (inv * g_ref[...]) + be_ref[...]

    return pl.pallas_call(
        body,
        grid=(_GRID,),
        in_specs=[
            pl.BlockSpec((_BLK, D), lambda i: (i, 0)),
            pl.BlockSpec((2, D), lambda i: (0, 0)),
            pl.BlockSpec((1, D), lambda i: (0, 0)),
            pl.BlockSpec((1, D), lambda i: (0, 0)),
        ],
        out_specs=pl.BlockSpec((_BLK, D), lambda i: (i, 0)),
        out_shape=jax.ShapeDtypeStruct((NPAD, D), jnp.float32),
    )(h, stats, gamma, beta)


def _cheb2(t0, t1, p4, W, b, xres):
    def body(t0_ref, t1_ref, p4a_ref, p4b_ref, w_ref, b_ref, x_ref, o_ref):
        t0v = t0_ref[...]
        t1v = t1_ref[...]
        t2v = 2.0 * (p4a_ref[...] + p4b_ref[...]) - t0v
        acc = (
            jnp.dot(t0v, w_ref[0], preferred_element_type=jnp.float32)
            + jnp.dot(t1v, w_ref[1], preferred_element_type=jnp.float32)
            + jnp.dot(t2v, w_ref[2], preferred_element_type=jnp.float32)
            + b_ref[...]
            + x_ref[...]
        )
        o_ref[...] = jnp.maximum(acc, 0.0)

    return pl.pallas_call(
        body,
        grid=(_GRID,),
        in_specs=[
            pl.BlockSpec((_BLK, D), lambda i: (i, 0)),
            pl.BlockSpec((_BLK, D), lambda i: (i, 0)),
            pl.BlockSpec((_BLK, D), lambda i: (i, 0)),
            pl.BlockSpec((_BLK, D), lambda i: (i + _GRID, 0)),
            pl.BlockSpec((3, D, D), lambda i: (0, 0, 0)),
            pl.BlockSpec((1, D), lambda i: (0, 0)),
            pl.BlockSpec((_BLK, D), lambda i: (i, 0)),
        ],
        out_specs=pl.BlockSpec((_BLK, D), lambda i: (i, 0)),
        out_shape=jax.ShapeDtypeStruct((NPAD, D), jnp.float32),
    )(t0, t1, p4, p4, W, b, xres)


def kernel(x, edge_index, edge_weight, W1, b1, gamma, beta, W2, b2):
    dst = edge_index[0]
    src = edge_index[1]
    pe = EP - E
    srcp = jnp.concatenate([src, jnp.zeros((pe,), jnp.int32)])
    dstp = jnp.concatenate([dst, jnp.full((pe,), N, jnp.int32)])
    wp = jnp.concatenate([edge_weight, jnp.zeros((pe,), jnp.float32)])
    src5 = srcp.reshape(NW, SPS, SCH, 1, CH)
    dst4 = dstp.reshape(NW, NCH, 1, CH)
    w5 = wp.reshape(NW, SPS, SCH, 1, CH)
    b1r = b1.reshape(1, D)
    b2r = b2.reshape(1, D)
    gr = gamma.reshape(1, D)
    br = beta.reshape(1, D)

    xp = jnp.pad(x, ((0, NPAD - N), (0, 0)))
    p1 = _spmv_partials(xp, src5, dst4, w5)
    tx1 = _combine(p1)
    p2 = _spmv_partials(tx1, src5, dst4, w5)
    h, stats = _cheb1(xp, tx1, p2, W1, b1r)
    y = _bn(h, stats, gr, br)
    p3 = _spmv_partials(y, src5, dst4, w5)
    ty1 = _combine(p3)
    p4 = _spmv_partials(ty1, src5, dst4, w5)
    out = _cheb2(y, ty1, p4, W2, b2r, xp)
    return out[:N]


# speedup vs baseline: 3.1044x; 1.1955x over previous
import functools

import jax
import jax.numpy as jnp
from jax import lax
from jax.experimental import pallas as pl
from jax.experimental.pallas import tpu as pltpu
from jax.experimental.pallas import tpu_sc as plsc

N = 10000
NPAD = 10240
E = 320000
D = 128
EPS = 1e-5

NC = 2
NS = 16
NW = NC * NS
CH = 128
EPWP = 10240
EP = EPWP * NW
NCH = EPWP // CH
NBUF = 2
PS = 8
NPASS = NCH // PS
RPT = NPAD // NS

_MESH = plsc.VectorSubcoreMesh(
    core_axis_name="c", subcore_axis_name="s", num_cores=NC, num_subcores=NS
)


def _spmv_partials(table, src5, dst4, w5):
    @functools.partial(
        pl.kernel,
        out_type=jax.ShapeDtypeStruct((NC * NPAD, D), jnp.float32),
        mesh=_MESH,
        scratch_types=[
            pltpu.VMEM((NCH, 1, CH), jnp.int32),
            pltpu.VMEM((2, PS, 1, CH), jnp.int32),
            pltpu.VMEM((2, PS, 1, CH), jnp.float32),
            pltpu.VMEM((NBUF, CH, D), jnp.float32),
            pltpu.VMEM_SHARED((NPAD, D), jnp.float32),
            pltpu.SemaphoreType.DMA,
            pltpu.SemaphoreType.DMA,
            pltpu.SemaphoreType.DMA,
            pltpu.SemaphoreType.DMA,
        ],
    )
    def k(table_hbm, src_hbm, dst_hbm, w_hbm, out_hbm,
          src_v, dstb, wb, rows_v, acc_sh, sem0, sem1, sem2, sem3):
        sems = (sem0, sem1, sem2, sem3)
        cid = lax.axis_index("c")
        sid = lax.axis_index("s")
        wid = sid * NC + cid

        c_src = pltpu.make_async_copy(src_hbm.at[wid], src_v, sems[3])
        c_src.start()
        pltpu.make_async_copy(w_hbm.at[wid, 0], wb.at[0], sems[2]).start()
        pltpu.make_async_copy(dst_hbm.at[wid, 0], dstb.at[0], sems[2]).start()

        zero16 = jnp.zeros((16,), jnp.float32)

        def zrow(i, carry):
            for q in range(D // 16):
                rows_v[0, i, pl.ds(q * 16, 16)] = zero16
            return carry

        lax.fori_loop(0, CH, zrow, 0)
        for t in range(RPT // CH):
            pltpu.sync_copy(rows_v.at[0], acc_sh.at[pl.ds(sid * RPT + t * CH, CH)])

        c_src.wait()

        for b in range(NBUF):
            pltpu.make_async_copy(
                table_hbm.at[src_v.at[b, 0]], rows_v.at[b], sems[b]
            ).start()

        plsc.subcore_barrier()

        def process(j, b, p, t):
            pltpu.make_async_copy(
                table_hbm.at[src_v.at[j, 0]], rows_v.at[b], sems[b]
            ).wait()

            def wmul(g, c2):
                wv = wb[p, t, 0, pl.ds(g * 16, 16)]
                for kk in range(16):
                    wr = wv[kk]
                    for q in range(D // 16):
                        rows_v[b, g * 16 + kk, pl.ds(q * 16, 16)] = (
                            rows_v[b, g * 16 + kk, pl.ds(q * 16, 16)] * wr
                        )
                return c2

            lax.fori_loop(0, CH // 16, wmul, 0)
            pltpu.sync_copy(rows_v.at[b], acc_sh.at[dstb.at[p, t, 0]], add=True)
            jn = jnp.minimum(j + NBUF, NCH - 1)
            pltpu.make_async_copy(
                table_hbm.at[src_v.at[jn, 0]], rows_v.at[b], sems[b]
            ).start()

        def run_pass(s, p):
            pltpu.make_async_copy(w_hbm.at[wid, s], wb.at[p], sems[2 + p]).wait()
            pltpu.make_async_copy(dst_hbm.at[wid, s], dstb.at[p], sems[2 + p]).wait()
            sn = jnp.minimum(s + 1, NPASS - 1)
            pltpu.make_async_copy(
                w_hbm.at[wid, sn], wb.at[1 - p], sems[2 + (1 - p)]
            ).start()
            pltpu.make_async_copy(
                dst_hbm.at[wid, sn], dstb.at[1 - p], sems[2 + (1 - p)]
            ).start()
            for t in range(PS):
                process(s * PS + t, t % NBUF, p, t)

        def outer(pp, carry):
            run_pass(2 * pp, 0)
            run_pass(2 * pp + 1, 1)
            return carry

        lax.fori_loop(0, NPASS // 2, outer, 0)

        for b in range(NBUF):
            pltpu.make_async_copy(
                table_hbm.at[src_v.at[NCH - 1, 0]], rows_v.at[b], sems[b]
            ).wait()
        pltpu.make_async_copy(w_hbm.at[wid, NPASS - 1], wb.at[0], sems[2]).wait()
        pltpu.make_async_copy(dst_hbm.at[wid, NPASS - 1], dstb.at[0], sems[2]).wait()

        plsc.subcore_barrier()
        pltpu.sync_copy(
            acc_sh.at[pl.ds(sid * RPT, RPT)],
            out_hbm.at[pl.ds(cid * NPAD + sid * RPT, RPT)],
        )

    return k(table, src5, dst4, w5)


_BLK = 2048
_GRID = NPAD // _BLK


def _combine(p):
    def body(a_ref, b_ref, o_ref):
        o_ref[...] = a_ref[...] + b_ref[...]

    return pl.pallas_call(
        body,
        grid=(_GRID,),
        in_specs=[
            pl.BlockSpec((_BLK, D), lambda i: (i, 0)),
            pl.BlockSpec((_BLK, D), lambda i: (i + _GRID, 0)),
        ],
        out_specs=pl.BlockSpec((_BLK, D), lambda i: (i, 0)),
        out_shape=jax.ShapeDtypeStruct((NPAD, D), jnp.float32),
    )(p, p)


def _cheb1(t0, t1, p2, W, b):
    def body(t0_ref, t1_ref, p2a_ref, p2b_ref, w_ref, b_ref, o_ref, st_ref):
        i = pl.program_id(0)
        t0v = t0_ref[...]
        t1v = t1_ref[...]
        t2v = 2.0 * (p2a_ref[...] + p2b_ref[...]) - t0v
        acc = (
            jnp.dot(t0v, w_ref[0], preferred_element_type=jnp.float32)
            + jnp.dot(t1v, w_ref[1], preferred_element_type=jnp.float32)
            + jnp.dot(t2v, w_ref[2], preferred_element_type=jnp.float32)
            + b_ref[...]
        )
        h = jnp.maximum(acc, 0.0)
        o_ref[...] = h
        rid = lax.broadcasted_iota(jnp.int32, (_BLK, 1), 0) + i * _BLK
        hm = jnp.where(rid < N, h, 0.0)
        ps = jnp.sum(hm, axis=0, keepdims=True)
        pq = jnp.sum(hm * hm, axis=0, keepdims=True)
        blk = jnp.concatenate([ps, pq], axis=0)

        @pl.when(i == 0)
        def _():
            st_ref[...] = blk

        @pl.when(i != 0)
        def _():
            st_ref[...] = st_ref[...] + blk

    return pl.pallas_call(
        body,
        grid=(_GRID,),
        in_specs=[
            pl.BlockSpec((_BLK, D), lambda i: (i, 0)),
            pl.BlockSpec((_BLK, D), lambda i: (i, 0)),
            pl.BlockSpec((_BLK, D), lambda i: (i, 0)),
            pl.BlockSpec((_BLK, D), lambda i: (i + _GRID, 0)),
            pl.BlockSpec((3, D, D), lambda i: (0, 0, 0)),
            pl.BlockSpec((1, D), lambda i: (0, 0)),
        ],
        out_specs=[
            pl.BlockSpec((_BLK, D), lambda i: (i, 0)),
            pl.BlockSpec((2, D), lambda i: (0, 0)),
        ],
        out_shape=[
            jax.ShapeDtypeStruct((NPAD, D), jnp.float32),
            jax.ShapeDtypeStruct((2, D), jnp.float32),
        ],
    )(t0, t1, p2, p2, W, b)


def _bn(h, stats, gamma, beta):
    def body(h_ref, st_ref, g_ref, be_ref, o_ref):
        mean = st_ref[0:1, :] * (1.0 / N)
        var = st_ref[1:2, :] * (1.0 / N) - mean * mean
        inv = lax.rsqrt(var + EPS)
        o_ref[...] = (h_ref[...] - mean) * (inv * g_ref[...]) + be_ref[...]

    return pl.pallas_call(
        body,
        grid=(_GRID,),
        in_specs=[
            pl.BlockSpec((_BLK, D), lambda i: (i, 0)),
            pl.BlockSpec((2, D), lambda i: (0, 0)),
            pl.BlockSpec((1, D), lambda i: (0, 0)),
            pl.BlockSpec((1, D), lambda i: (0, 0)),
        ],
        out_specs=pl.BlockSpec((_BLK, D), lambda i: (i, 0)),
        out_shape=jax.ShapeDtypeStruct((NPAD, D), jnp.float32),
    )(h, stats, gamma, beta)


def _cheb2(t0, t1, p4, W, b, xres):
    def body(t0_ref, t1_ref, p4a_ref, p4b_ref, w_ref, b_ref, x_ref, o_ref):
        t0v = t0_ref[...]
        t1v = t1_ref[...]
        t2v = 2.0 * (p4a_ref[...] + p4b_ref[...]) - t0v
        acc = (
            jnp.dot(t0v, w_ref[0], preferred_element_type=jnp.float32)
            + jnp.dot(t1v, w_ref[1], preferred_element_type=jnp.float32)
            + jnp.dot(t2v, w_ref[2], preferred_element_type=jnp.float32)
            + b_ref[...]
            + x_ref[...]
        )
        o_ref[...] = jnp.maximum(acc, 0.0)

    return pl.pallas_call(
        body,
        grid=(_GRID,),
        in_specs=[
            pl.BlockSpec((_BLK, D), lambda i: (i, 0)),
            pl.BlockSpec((_BLK, D), lambda i: (i, 0)),
            pl.BlockSpec((_BLK, D), lambda i: (i, 0)),
            pl.BlockSpec((_BLK, D), lambda i: (i + _GRID, 0)),
            pl.BlockSpec((3, D, D), lambda i: (0, 0, 0)),
            pl.BlockSpec((1, D), lambda i: (0, 0)),
            pl.BlockSpec((_BLK, D), lambda i: (i, 0)),
        ],
        out_specs=pl.BlockSpec((_BLK, D), lambda i: (i, 0)),
        out_shape=jax.ShapeDtypeStruct((NPAD, D), jnp.float32),
    )(t0, t1, p4, p4, W, b, xres)


def kernel(x, edge_index, edge_weight, W1, b1, gamma, beta, W2, b2):
    dst = edge_index[0]
    src = edge_index[1]
    pe = EP - E
    srcp = jnp.concatenate([src, jnp.zeros((pe,), jnp.int32)])
    dstp = jnp.concatenate([dst, jnp.full((pe,), N, jnp.int32)])
    wp = jnp.concatenate([edge_weight, jnp.zeros((pe,), jnp.float32)])
    src5 = srcp.reshape(NW, NCH, 1, CH)
    dst4 = dstp.reshape(NW, NPASS, PS, 1, CH)
    w5 = wp.reshape(NW, NPASS, PS, 1, CH)
    b1r = b1.reshape(1, D)
    b2r = b2.reshape(1, D)
    gr = gamma.reshape(1, D)
    br = beta.reshape(1, D)

    xp = jnp.pad(x, ((0, NPAD - N), (0, 0)))
    p1 = _spmv_partials(xp, src5, dst4, w5)
    tx1 = _combine(p1)
    p2 = _spmv_partials(tx1, src5, dst4, w5)
    h, stats = _cheb1(xp, tx1, p2, W1, b1r)
    y = _bn(h, stats, gr, br)
    p3 = _spmv_partials(y, src5, dst4, w5)
    ty1 = _combine(p3)
    p4 = _spmv_partials(ty1, src5, dst4, w5)
    out = _cheb2(y, ty1, p4, W2, b2r, xp)
    return out[:N]


# speedup vs baseline: 3.2179x; 1.0366x over previous
import functools

import jax
import jax.numpy as jnp
from jax import lax
from jax.experimental import pallas as pl
from jax.experimental.pallas import tpu as pltpu
from jax.experimental.pallas import tpu_sc as plsc

N = 10000
NPAD = 10240
E = 320000
D = 128
EPS = 1e-5

NC = 2
NS = 16
NW = NC * NS
CH = 128
EP = 2560 * CH
NCH0 = 124
NCH1 = 36
NCHMAX = max(NCH0, NCH1)
NBUF = 2
PS = 2
NP0 = NCH0 // PS
NP1 = NCH1 // PS
NACC = 10112
RPTA = NACC // NS

_MESH = plsc.VectorSubcoreMesh(
    core_axis_name="c", subcore_axis_name="s", num_cores=NC, num_subcores=NS
)


def _spmv_partials(table, src5, dst4, w5):
    @functools.partial(
        pl.kernel,
        out_type=jax.ShapeDtypeStruct((NC * NPAD, D), jnp.float32),
        mesh=_MESH,
        scratch_types=[
            pltpu.VMEM((NCHMAX, 1, CH), jnp.int32),
            pltpu.VMEM((2, PS, 1, CH), jnp.int32),
            pltpu.VMEM((2, PS, 1, CH), jnp.float32),
            pltpu.VMEM((NBUF, CH, D), jnp.float32),
            pltpu.VMEM_SHARED((NACC, D), jnp.float32),
            pltpu.SemaphoreType.DMA,
            pltpu.SemaphoreType.DMA,
            pltpu.SemaphoreType.DMA,
            pltpu.SemaphoreType.DMA,
        ],
    )
    def k(table_hbm, src_hbm, dst_hbm, w_hbm, out_hbm,
          src_v, dstb, wb, rows_v, acc_sh, sem0, sem1, sem2, sem3):
        sems = (sem0, sem1, sem2, sem3)
        cid = lax.axis_index("c")
        sid = lax.axis_index("s")
        is0 = cid == 0
        nch = jnp.where(is0, NCH0, NCH1)
        npass = jnp.where(is0, NP0, NP1)
        cbase = jnp.where(is0, sid * NCH0, NS * NCH0 + sid * NCH1)
        pbase = cbase // PS

        semS = sems[2]
        c_src = pltpu.make_async_copy(
            src_hbm.at[pl.ds(cbase, NCHMAX)], src_v, sems[3]
        )
        c_src.start()
        pltpu.make_async_copy(w_hbm.at[pbase], wb.at[0], semS).start()
        pltpu.make_async_copy(dst_hbm.at[pbase], dstb.at[0], semS).start()

        zero16 = jnp.zeros((16,), jnp.float32)

        def zrow(i, carry):
            for q in range(D // 16):
                rows_v[0, i, pl.ds(q * 16, 16)] = zero16
            return carry

        lax.fori_loop(0, CH, zrow, 0)
        for t in range(RPTA // CH):
            pltpu.sync_copy(rows_v.at[0], acc_sh.at[pl.ds(sid * RPTA + t * CH, CH)])
        _REM = RPTA - (RPTA // CH) * CH
        pltpu.sync_copy(
            rows_v.at[0, pl.ds(0, _REM)],
            acc_sh.at[pl.ds(sid * RPTA + (RPTA // CH) * CH, _REM)],
        )

        c_src.wait()

        for b in range(NBUF):
            pltpu.make_async_copy(
                table_hbm.at[src_v.at[b, 0]], rows_v.at[b], sems[b]
            ).start()

        plsc.subcore_barrier()

        def process(j, b, p):
            pltpu.make_async_copy(
                table_hbm.at[src_v.at[j, 0]], rows_v.at[b], sems[b]
            ).wait()

            def wmul(g, c2):
                wv = wb[p, b, 0, pl.ds(g * 16, 16)]
                for kk in range(16):
                    wr = wv[kk]
                    for q in range(D // 16):
                        rows_v[b, g * 16 + kk, pl.ds(q * 16, 16)] = (
                            rows_v[b, g * 16 + kk, pl.ds(q * 16, 16)] * wr
                        )
                return c2

            lax.fori_loop(0, CH // 16, wmul, 0)
            pltpu.sync_copy(rows_v.at[b], acc_sh.at[dstb.at[p, b, 0]], add=True)
            jn = jnp.minimum(j + NBUF, nch - 1)
            pltpu.make_async_copy(
                table_hbm.at[src_v.at[jn, 0]], rows_v.at[b], sems[b]
            ).start()

        def run_pass(s, carry):
            p = s % 2
            pltpu.make_async_copy(w_hbm.at[pbase + s], wb.at[p], semS).wait()
            pltpu.make_async_copy(dst_hbm.at[pbase + s], dstb.at[p], semS).wait()
            sn = jnp.minimum(s + 1, npass - 1)
            pltpu.make_async_copy(w_hbm.at[pbase + sn], wb.at[1 - p], semS).start()
            pltpu.make_async_copy(dst_hbm.at[pbase + sn], dstb.at[1 - p], semS).start()
            for b in range(PS):
                process(s * PS + b, b, p)
            return carry

        lax.fori_loop(0, npass, run_pass, 0)

        for b in range(NBUF):
            pltpu.make_async_copy(
                table_hbm.at[src_v.at[nch - 1, 0]], rows_v.at[b], sems[b]
            ).wait()
        pltpu.make_async_copy(w_hbm.at[pbase], wb.at[0], semS).wait()
        pltpu.make_async_copy(dst_hbm.at[pbase], dstb.at[0], semS).wait()

        plsc.subcore_barrier()
        pltpu.sync_copy(
            acc_sh.at[pl.ds(sid * RPTA, RPTA)],
            out_hbm.at[pl.ds(cid * NPAD + sid * RPTA, RPTA)],
        )

    return k(table, src5, dst4, w5)


_BLK = 2048
_GRID = NPAD // _BLK


def _combine(p):
    def body(a_ref, b_ref, o_ref):
        o_ref[...] = a_ref[...] + b_ref[...]

    return pl.pallas_call(
        body,
        grid=(_GRID,),
        in_specs=[
            pl.BlockSpec((_BLK, D), lambda i: (i, 0)),
            pl.BlockSpec((_BLK, D), lambda i: (i + _GRID, 0)),
        ],
        out_specs=pl.BlockSpec((_BLK, D), lambda i: (i, 0)),
        out_shape=jax.ShapeDtypeStruct((NPAD, D), jnp.float32),
    )(p, p)


def _cheb1(t0, t1, p2, W, b):
    def body(t0_ref, t1_ref, p2a_ref, p2b_ref, w_ref, b_ref, o_ref, st_ref):
        i = pl.program_id(0)
        t0v = t0_ref[...]
        t1v = t1_ref[...]
        t2v = 2.0 * (p2a_ref[...] + p2b_ref[...]) - t0v
        acc = (
            jnp.dot(t0v, w_ref[0], preferred_element_type=jnp.float32)
            + jnp.dot(t1v, w_ref[1], preferred_element_type=jnp.float32)
            + jnp.dot(t2v, w_ref[2], preferred_element_type=jnp.float32)
            + b_ref[...]
        )
        h = jnp.maximum(acc, 0.0)
        o_ref[...] = h
        rid = lax.broadcasted_iota(jnp.int32, (_BLK, 1), 0) + i * _BLK
        hm = jnp.where(rid < N, h, 0.0)
        ps = jnp.sum(hm, axis=0, keepdims=True)
        pq = jnp.sum(hm * hm, axis=0, keepdims=True)
        blk = jnp.concatenate([ps, pq], axis=0)

        @pl.when(i == 0)
        def _():
            st_ref[...] = blk

        @pl.when(i != 0)
        def _():
            st_ref[...] = st_ref[...] + blk

    return pl.pallas_call(
        body,
        grid=(_GRID,),
        in_specs=[
            pl.BlockSpec((_BLK, D), lambda i: (i, 0)),
            pl.BlockSpec((_BLK, D), lambda i: (i, 0)),
            pl.BlockSpec((_BLK, D), lambda i: (i, 0)),
            pl.BlockSpec((_BLK, D), lambda i: (i + _GRID, 0)),
            pl.BlockSpec((3, D, D), lambda i: (0, 0, 0)),
            pl.BlockSpec((1, D), lambda i: (0, 0)),
        ],
        out_specs=[
            pl.BlockSpec((_BLK, D), lambda i: (i, 0)),
            pl.BlockSpec((2, D), lambda i: (0, 0)),
        ],
        out_shape=[
            jax.ShapeDtypeStruct((NPAD, D), jnp.float32),
            jax.ShapeDtypeStruct((2, D), jnp.float32),
        ],
    )(t0, t1, p2, p2, W, b)


def _bn(h, stats, gamma, beta):
    def body(h_ref, st_ref, g_ref, be_ref, o_ref):
        mean = st_ref[0:1, :] * (1.0 / N)
        var = st_ref[1:2, :] * (1.0 / N) - mean * mean
        inv = lax.rsqrt(var + EPS)
        o_ref[...] = (h_ref[...] - mean) * (inv * g_ref[...]) + be_ref[...]

    return pl.pallas_call(
        body,
        grid=(_GRID,),
        in_specs=[
            pl.BlockSpec((_BLK, D), lambda i: (i, 0)),
            pl.BlockSpec((2, D), lambda i: (0, 0)),
            pl.BlockSpec((1, D), lambda i: (0, 0)),
            pl.BlockSpec((1, D), lambda i: (0, 0)),
        ],
        out_specs=pl.BlockSpec((_BLK, D), lambda i: (i, 0)),
        out_shape=jax.ShapeDtypeStruct((NPAD, D), jnp.float32),
    )(h, stats, gamma, beta)


def _cheb2(t0, t1, p4, W, b, xres):
    def body(t0_ref, t1_ref, p4a_ref, p4b_ref, w_ref, b_ref, x_ref, o_ref):
        t0v = t0_ref[...]
        t1v = t1_ref[...]
        t2v = 2.0 * (p4a_ref[...] + p4b_ref[...]) - t0v
        acc = (
            jnp.dot(t0v, w_ref[0], preferred_element_type=jnp.float32)
            + jnp.dot(t1v, w_ref[1], preferred_element_type=jnp.float32)
            + jnp.dot(t2v, w_ref[2], preferred_element_type=jnp.float32)
            + b_ref[...]
            + x_ref[...]
        )
        o_ref[...] = jnp.maximum(acc, 0.0)

    return pl.pallas_call(
        body,
        grid=(_GRID,),
        in_specs=[
            pl.BlockSpec((_BLK, D), lambda i: (i, 0)),
            pl.BlockSpec((_BLK, D), lambda i: (i, 0)),
            pl.BlockSpec((_BLK, D), lambda i: (i, 0)),
            pl.BlockSpec((_BLK, D), lambda i: (i + _GRID, 0)),
            pl.BlockSpec((3, D, D), lambda i: (0, 0, 0)),
            pl.BlockSpec((1, D), lambda i: (0, 0)),
            pl.BlockSpec((_BLK, D), lambda i: (i, 0)),
        ],
        out_specs=pl.BlockSpec((_BLK, D), lambda i: (i, 0)),
        out_shape=jax.ShapeDtypeStruct((NPAD, D), jnp.float32),
    )(t0, t1, p4, p4, W, b, xres)


def kernel(x, edge_index, edge_weight, W1, b1, gamma, beta, W2, b2):
    dst = edge_index[0]
    src = edge_index[1]
    pe = EP - E
    srcp = jnp.concatenate([src, jnp.zeros((pe + NCHMAX * CH,), jnp.int32)])
    dstp = jnp.concatenate([dst, jnp.full((pe,), N, jnp.int32)])
    wp = jnp.concatenate([edge_weight, jnp.zeros((pe,), jnp.float32)])
    src5 = srcp.reshape(EP // CH + NCHMAX, 1, CH)
    dst4 = dstp.reshape(EP // (PS * CH), PS, 1, CH)
    w5 = wp.reshape(EP // (PS * CH), PS, 1, CH)
    b1r = b1.reshape(1, D)
    b2r = b2.reshape(1, D)
    gr = gamma.reshape(1, D)
    br = beta.reshape(1, D)

    xp = jnp.pad(x, ((0, NPAD - N), (0, 0)))
    p1 = _spmv_partials(xp, src5, dst4, w5)
    tx1 = _combine(p1)
    p2 = _spmv_partials(tx1, src5, dst4, w5)
    h, stats = _cheb1(xp, tx1, p2, W1, b1r)
    y = _bn(h, stats, gr, br)
    p3 = _spmv_partials(y, src5, dst4, w5)
    ty1 = _combine(p3)
    p4 = _spmv_partials(ty1, src5, dst4, w5)
    out = _cheb2(y, ty1, p4, W2, b2r, xp)
    return out[:N]
